# baseline (speedup 1.0000x reference)
import functools

import jax
import jax.numpy as jnp
from jax import lax
from jax.experimental import pallas as pl
from jax.experimental.pallas import tpu as pltpu
from jax.experimental.pallas import tpu_sc as plsc

N = 10000
D = 128
E = 320000
NC = 2
NS = 16
NW = NC * NS

NPAD = 10240
ROWS_PER_TILE = NPAD // NS
K = 80
CH0 = 158
CH1 = 94
CHMAX = max(CH0, CH1)
EPT0 = K * CH0
EPT1 = K * CH1
EPAD = NS * (EPT0 + EPT1)
ZK = 160

def _hist_body(dst_hbm, out_hbm, idx_v, hist_v, hsum, tb, res_v):
    cid = lax.axis_index("c")
    sid = lax.axis_index("s")
    wid = cid * NS + sid

    pltpu.sync_copy(dst_hbm.at[wid], idx_v)

    zv = jnp.full((16,), 0.0, jnp.float32)

    def zrow(r, _):
        hist_v[pl.ds(r * 16, 16)] = zv
        return 0

    lax.fori_loop(0, NPAD // 16, zrow, 0)

    ones = jnp.full((16,), 1.0, jnp.float32)

    def chunk(j, _):
        for k in range(K // 16):
            idx = idx_v[j, pl.ds(k * 16, 16)]
            plsc.addupdate_scatter(hist_v, [idx], ones)
        return 0

    @pl.when(cid == 0)
    def _():
        lax.fori_loop(0, CH0, chunk, 0)

    @pl.when(cid != 0)
    def _():
        lax.fori_loop(0, CH1, chunk, 0)

    pltpu.sync_copy(hist_v, hsum.at[sid])
    plsc.subcore_barrier()

    base = sid * ROWS_PER_TILE
    pltpu.sync_copy(hsum.at[:, pl.ds(base, ROWS_PER_TILE)], tb)

    def red(c, _):
        acc = tb[0, pl.ds(c * 16, 16)]
        for t in range(1, NS):
            acc = acc + tb[t, pl.ds(c * 16, 16)]
        res_v[pl.ds(c * 16, 16)] = acc
        return 0

    lax.fori_loop(0, ROWS_PER_TILE // 16, red, 0)
    pltpu.sync_copy(res_v, out_hbm.at[cid, pl.ds(base, ROWS_PER_TILE)])


@functools.cache
def _hist_kernel():
    mesh = plsc.VectorSubcoreMesh(core_axis_name="c", subcore_axis_name="s")
    return pl.kernel(
        _hist_body,
        out_type=jax.ShapeDtypeStruct((NC, NPAD), jnp.float32),
        mesh=mesh,
        compiler_params=pltpu.CompilerParams(needs_layout_passes=False, use_tc_tiling_on_sc=False),
        scratch_types=[
            pltpu.VMEM((CHMAX, K), jnp.int32),
            pltpu.VMEM((NPAD,), jnp.float32),
            pltpu.VMEM_SHARED((NS, NPAD), jnp.float32),
            pltpu.VMEM((NS, ROWS_PER_TILE), jnp.float32),
            pltpu.VMEM((ROWS_PER_TILE,), jnp.float32),
        ],
    )


def _edge_body(y_hbm, src_hbm, dst_hbm, out_hbm,
               src_v, dst_v, rows, acc, gsems):
    cid = lax.axis_index("c")
    sid = lax.axis_index("s")
    wid = cid * NS + sid

    pltpu.sync_copy(src_hbm.at[wid], src_v)
    pltpu.sync_copy(dst_hbm.at[wid], dst_v)

    zv = jnp.full((16,), 0.0, jnp.float32)

    def zrow(r, _):
        for c in range(D // 16):
            rows[0, r, pl.ds(c * 16, 16)] = zv
        return 0

    lax.fori_loop(0, K, zrow, 0)
    base = sid * ROWS_PER_TILE
    for s in range(ROWS_PER_TILE // K):
        pltpu.sync_copy(rows.at[0], acc.at[pl.ds(base + s * K, K)])
    plsc.subcore_barrier()

    pltpu.async_copy(y_hbm.at[src_v.at[0]], rows.at[0], gsems.at[0])

    def run(nch):
        def step(i, _):
            j = 2 * i
            pltpu.make_async_copy(y_hbm.at[src_v.at[j]], rows.at[0],
                                  gsems.at[0]).wait()
            pltpu.async_copy(y_hbm.at[src_v.at[j + 1]], rows.at[1],
                             gsems.at[1])
            pltpu.sync_copy(rows.at[0], acc.at[dst_v.at[j]], add=True)
            pltpu.make_async_copy(y_hbm.at[src_v.at[j + 1]], rows.at[1],
                                  gsems.at[1]).wait()

            @pl.when(i < nch // 2 - 1)
            def _():
                pltpu.async_copy(y_hbm.at[src_v.at[j + 2]], rows.at[0],
                                 gsems.at[0])

            pltpu.sync_copy(rows.at[1], acc.at[dst_v.at[j + 1]], add=True)
            return 0

        lax.fori_loop(0, nch // 2, step, 0)

    @pl.when(cid == 0)
    def _():
        run(CH0)

    @pl.when(cid != 0)
    def _():
        run(CH1)

    plsc.subcore_barrier()
    pltpu.sync_copy(acc.at[pl.ds(base, ROWS_PER_TILE)],
                    out_hbm.at[cid, pl.ds(base, ROWS_PER_TILE)])


@functools.cache
def _edge_kernel():
    mesh = plsc.VectorSubcoreMesh(core_axis_name="c", subcore_axis_name="s")
    return pl.kernel(
        _edge_body,
        out_type=jax.ShapeDtypeStruct((NC, NPAD, D), jnp.float32),
        mesh=mesh,
        compiler_params=pltpu.CompilerParams(needs_layout_passes=False, use_tc_tiling_on_sc=False),
        scratch_types=[
            pltpu.VMEM((CHMAX, K), jnp.int32),
            pltpu.VMEM((CHMAX, K), jnp.int32),
            pltpu.VMEM((2, K, D), jnp.float32),
            pltpu.VMEM_SHARED((NPAD, D), jnp.float32),
            pltpu.SemaphoreType.DMA((2,)),
        ],
    )


def _prep_body(state_ref, w_ref, hist_ref, y_ref, dinv_ref):
    deg = hist_ref[0] + hist_ref[1] + 1.0
    dinv = lax.rsqrt(deg)[:N]
    dinv_ref[...] = dinv
    xw = jnp.dot(state_ref[...], w_ref[...], preferred_element_type=jnp.float32)
    y_ref[...] = xw * dinv


def _tc_prep(state, w_conv, hist3):
    return pl.pallas_call(
        _prep_body,
        out_shape=[
            jax.ShapeDtypeStruct((N, D), jnp.float32),
            jax.ShapeDtypeStruct((N, 1), jnp.float32),
        ],
    )(state, w_conv, hist3)


def _final_body(agg_ref, y_ref, dinv_ref, state_ref, bc_ref,
                w1_ref, b1_ref, w2_ref, b2_ref, w3_ref, b3_ref, out_ref):
    agg = agg_ref[0, :N, :] + agg_ref[1, :N, :]
    conv = dinv_ref[...] * (agg + y_ref[...]) + bc_ref[...]
    h = jnp.maximum(conv, 0.0) + state_ref[...]
    z = jnp.dot(h, w1_ref[...], preferred_element_type=jnp.float32) + b1_ref[...]
    z = jnp.where(z >= 0.0, z, 0.01 * z)
    z = jnp.dot(z, w2_ref[...], preferred_element_type=jnp.float32) + b2_ref[...]
    z = jnp.where(z >= 0.0, z, 0.01 * z)
    t = jnp.sum(z * w3_ref[...], axis=1, keepdims=True) + b3_ref[...]
    c = jnp.maximum(t, 0.0) + jnp.log1p(jnp.exp(-jnp.abs(t)))
    out_ref[...] = c / (jnp.sum(c) + 1e-20)


def _tc_final(agg2, y, dinv, state, b_conv, W1, b1, W2, b2, w3r, b3):
    return pl.pallas_call(
        _final_body,
        out_shape=jax.ShapeDtypeStruct((N, 1), jnp.float32),
    )(agg2, y, dinv, state, b_conv, W1, b1, W2, b2, w3r, b3)


def kernel(state, edge_index, W_conv, b_conv, W1, b1, W2, b2, W3, b3,
           deterministic=True):
    npad_e = EPAD - E
    src_f = jnp.concatenate([edge_index[0], jnp.zeros((npad_e,), jnp.int32)])
    dst_f = jnp.concatenate([edge_index[1],
                             jnp.full((npad_e,), NPAD - 1, jnp.int32)])

    def split(flat, fill):
        e0 = flat[:NS * EPT0].reshape(NS, CH0, K)
        e1 = flat[NS * EPT0:].reshape(NS, CH1, K)
        if CH0 < CHMAX:
            pad = jnp.full((NS, CHMAX - CH0, K), fill, jnp.int32)
            e0 = jnp.concatenate([e0, pad], axis=1)
        if CH1 < CHMAX:
            pad = jnp.full((NS, CHMAX - CH1, K), fill, jnp.int32)
            e1 = jnp.concatenate([e1, pad], axis=1)
        return jnp.concatenate([e0, e1], axis=0)

    src = split(src_f, 0)
    dst = split(dst_f, NPAD - 1)

    hist = _hist_kernel()(dst)
    hist3 = hist.reshape(NC, NPAD, 1)
    y, dinv = _tc_prep(state, W_conv, hist3)
    agg2 = _edge_kernel()(y, src, dst)
    action = _tc_final(agg2, y, dinv, state,
                       b_conv.reshape(1, D),
                       W1, b1.reshape(1, -1), W2, b2.reshape(1, -1),
                       W3.reshape(1, -1), b3.reshape(1, 1))
    return action.reshape(N // 10, 10)


# speedup vs baseline: 1.2044x; 1.2044x over previous
import functools

import jax
import jax.numpy as jnp
from jax import lax
from jax.experimental import pallas as pl
from jax.experimental.pallas import tpu as pltpu
from jax.experimental.pallas import tpu_sc as plsc

N = 10000
D = 128
E = 320000
NC = 2
NS = 16
NW = NC * NS

NPAD = 10240
ROWS_PER_TILE = NPAD // NS
K = 80
NCH_TOTAL = E // K
CH0 = 166
CH1 = 84
assert NS * (CH0 + CH1) == NCH_TOTAL
CHMAX = max(CH0, CH1)

def _hist_body(dst_hbm, out_hbm, idx_v, hist_v, hsum, tb, res_v):
    cid = lax.axis_index("c")
    sid = lax.axis_index("s")

    @pl.when(cid == 0)
    def _():
        pltpu.sync_copy(dst_hbm.at[pl.ds(sid * CH0, CH0)],
                        idx_v.at[pl.ds(0, CH0)])

    @pl.when(cid != 0)
    def _():
        pltpu.sync_copy(dst_hbm.at[pl.ds(NS * CH0 + sid * CH1, CH1)],
                        idx_v.at[pl.ds(0, CH1)])

    zv = jnp.full((16,), 0.0, jnp.float32)

    def zrow(r, _):
        hist_v[pl.ds(r * 16, 16)] = zv
        return 0

    lax.fori_loop(0, NPAD // 16, zrow, 0)

    ones = jnp.full((16,), 1.0, jnp.float32)

    def chunk(j, _):
        for k in range(K // 16):
            idx = idx_v[j, pl.ds(k * 16, 16)]
            plsc.addupdate_scatter(hist_v, [idx], ones)
        return 0

    @pl.when(cid == 0)
    def _():
        lax.fori_loop(0, CH0, chunk, 0)

    @pl.when(cid != 0)
    def _():
        lax.fori_loop(0, CH1, chunk, 0)

    pltpu.sync_copy(hist_v, hsum.at[sid])
    plsc.subcore_barrier()

    base = sid * ROWS_PER_TILE
    pltpu.sync_copy(hsum.at[:, pl.ds(base, ROWS_PER_TILE)], tb)

    def red(c, _):
        acc = tb[0, pl.ds(c * 16, 16)]
        for t in range(1, NS):
            acc = acc + tb[t, pl.ds(c * 16, 16)]
        res_v[pl.ds(c * 16, 16)] = acc
        return 0

    lax.fori_loop(0, ROWS_PER_TILE // 16, red, 0)
    pltpu.sync_copy(res_v, out_hbm.at[cid, pl.ds(base, ROWS_PER_TILE)])


@functools.cache
def _hist_kernel():
    mesh = plsc.VectorSubcoreMesh(core_axis_name="c", subcore_axis_name="s")
    return pl.kernel(
        _hist_body,
        out_type=jax.ShapeDtypeStruct((NC, NPAD), jnp.float32),
        mesh=mesh,
        compiler_params=pltpu.CompilerParams(needs_layout_passes=False, use_tc_tiling_on_sc=False),
        scratch_types=[
            pltpu.VMEM((CHMAX, K), jnp.int32),
            pltpu.VMEM((NPAD,), jnp.float32),
            pltpu.VMEM_SHARED((NS, NPAD), jnp.float32),
            pltpu.VMEM((NS, ROWS_PER_TILE), jnp.float32),
            pltpu.VMEM((ROWS_PER_TILE,), jnp.float32),
        ],
    )


def _edge_body(y_hbm, src_hbm, dst_hbm, out_hbm,
               src_v, dst_v, rows, acc, gsems):
    cid = lax.axis_index("c")
    sid = lax.axis_index("s")

    @pl.when(cid == 0)
    def _():
        start = sid * CH0
        pltpu.sync_copy(src_hbm.at[pl.ds(start, CH0)],
                        src_v.at[pl.ds(0, CH0)])
        pltpu.sync_copy(dst_hbm.at[pl.ds(start, CH0)],
                        dst_v.at[pl.ds(0, CH0)])

    @pl.when(cid != 0)
    def _():
        start = NS * CH0 + sid * CH1
        pltpu.sync_copy(src_hbm.at[pl.ds(start, CH1)],
                        src_v.at[pl.ds(0, CH1)])
        pltpu.sync_copy(dst_hbm.at[pl.ds(start, CH1)],
                        dst_v.at[pl.ds(0, CH1)])

    zv = jnp.full((16,), 0.0, jnp.float32)

    def zrow(r, _):
        for c in range(D // 16):
            rows[0, r, pl.ds(c * 16, 16)] = zv
        return 0

    lax.fori_loop(0, K, zrow, 0)
    base = sid * ROWS_PER_TILE
    for s in range(ROWS_PER_TILE // K):
        pltpu.sync_copy(rows.at[0], acc.at[pl.ds(base + s * K, K)])
    plsc.subcore_barrier()

    pltpu.async_copy(y_hbm.at[src_v.at[0]], rows.at[0], gsems.at[0])

    def run(nch):
        def step(i, _):
            j = 2 * i
            pltpu.make_async_copy(y_hbm.at[src_v.at[j]], rows.at[0],
                                  gsems.at[0]).wait()
            pltpu.async_copy(y_hbm.at[src_v.at[j + 1]], rows.at[1],
                             gsems.at[1])
            pltpu.sync_copy(rows.at[0], acc.at[dst_v.at[j]], add=True)
            pltpu.make_async_copy(y_hbm.at[src_v.at[j + 1]], rows.at[1],
                                  gsems.at[1]).wait()

            @pl.when(i < nch // 2 - 1)
            def _():
                pltpu.async_copy(y_hbm.at[src_v.at[j + 2]], rows.at[0],
                                 gsems.at[0])

            pltpu.sync_copy(rows.at[1], acc.at[dst_v.at[j + 1]], add=True)
            return 0

        lax.fori_loop(0, nch // 2, step, 0)

    @pl.when(cid == 0)
    def _():
        run(CH0)

    @pl.when(cid != 0)
    def _():
        run(CH1)

    plsc.subcore_barrier()
    pltpu.sync_copy(acc.at[pl.ds(base, ROWS_PER_TILE)],
                    out_hbm.at[cid, pl.ds(base, ROWS_PER_TILE)])


@functools.cache
def _edge_kernel():
    mesh = plsc.VectorSubcoreMesh(core_axis_name="c", subcore_axis_name="s")
    return pl.kernel(
        _edge_body,
        out_type=jax.ShapeDtypeStruct((NC, NPAD, D), jnp.float32),
        mesh=mesh,
        compiler_params=pltpu.CompilerParams(needs_layout_passes=False, use_tc_tiling_on_sc=False),
        scratch_types=[
            pltpu.VMEM((CHMAX, K), jnp.int32),
            pltpu.VMEM((CHMAX, K), jnp.int32),
            pltpu.VMEM((2, K, D), jnp.float32),
            pltpu.VMEM_SHARED((NPAD, D), jnp.float32),
            pltpu.SemaphoreType.DMA((2,)),
        ],
    )


def _prep_body(state_ref, w_ref, hist_ref, y_ref, dinv_ref):
    deg = hist_ref[0, :] + hist_ref[1, :] + 1.0
    dinv = lax.rsqrt(deg)[:N].reshape(N, 1)
    dinv_ref[...] = dinv
    xw = jnp.dot(state_ref[...], w_ref[...], preferred_element_type=jnp.float32)
    y_ref[...] = xw * dinv


def _tc_prep(state, w_conv, hist):
    return pl.pallas_call(
        _prep_body,
        out_shape=[
            jax.ShapeDtypeStruct((N, D), jnp.float32),
            jax.ShapeDtypeStruct((N, 1), jnp.float32),
        ],
    )(state, w_conv, hist)


def _final_body(agg_ref, y_ref, dinv_ref, state_ref, bc_ref,
                w1_ref, b1_ref, w2_ref, b2_ref, w3_ref, b3_ref, out_ref):
    agg = agg_ref[0, :N, :] + agg_ref[1, :N, :]
    conv = dinv_ref[...] * (agg + y_ref[...]) + bc_ref[...]
    h = jnp.maximum(conv, 0.0) + state_ref[...]
    z = jnp.dot(h, w1_ref[...], preferred_element_type=jnp.float32) + b1_ref[...]
    z = jnp.where(z >= 0.0, z, 0.01 * z)
    z = jnp.dot(z, w2_ref[...], preferred_element_type=jnp.float32) + b2_ref[...]
    z = jnp.where(z >= 0.0, z, 0.01 * z)
    t = jnp.sum(z * w3_ref[...], axis=1, keepdims=True) + b3_ref[...]
    c = jnp.maximum(t, 0.0) + jnp.log1p(jnp.exp(-jnp.abs(t)))
    out_ref[...] = c / (jnp.sum(c) + 1e-20)


def _tc_final(agg2, y, dinv, state, b_conv, W1, b1, W2, b2, w3r, b3):
    return pl.pallas_call(
        _final_body,
        out_shape=jax.ShapeDtypeStruct((N, 1), jnp.float32),
    )(agg2, y, dinv, state, b_conv, W1, b1, W2, b2, w3r, b3)


def kernel(state, edge_index, W_conv, b_conv, W1, b1, W2, b2, W3, b3,
           deterministic=True):
    src = edge_index[0].reshape(NCH_TOTAL, K)
    dst = edge_index[1].reshape(NCH_TOTAL, K)

    hist = _hist_kernel()(dst)
    y, dinv = _tc_prep(state, W_conv, hist)
    agg2 = _edge_kernel()(y, src, dst)
    action = _tc_final(agg2, y, dinv, state,
                       b_conv.reshape(1, D),
                       W1, b1.reshape(1, -1), W2, b2.reshape(1, -1),
                       W3.reshape(1, -1), b3.reshape(1, 1))
    return action.reshape(N // 10, 10)


# speedup vs baseline: 1.5173x; 1.2598x over previous
import functools

import jax
import jax.numpy as jnp
from jax import lax
from jax.experimental import pallas as pl
from jax.experimental.pallas import tpu as pltpu
from jax.experimental.pallas import tpu_sc as plsc

N = 10000
D = 128
E = 320000
NC = 2
NS = 16
NW = NC * NS

NPAD = 10240
ROWS_PER_TILE = NPAD // NS
K = 80
NCH_TOTAL = E // K
CH0 = 126
CH1 = 124
assert NS * (CH0 + CH1) == NCH_TOTAL
CHMAX = max(CH0, CH1)

def _hist_body(e_hbm, out_hbm, idx_v, hist_v, hsum, tb, res_v):
    cid = lax.axis_index("c")
    sid = lax.axis_index("s")

    @pl.when(cid == 0)
    def _():
        pltpu.sync_copy(e_hbm.at[1, pl.ds(sid * CH0, CH0)],
                        idx_v.at[pl.ds(0, CH0)])

    @pl.when(cid != 0)
    def _():
        pltpu.sync_copy(e_hbm.at[1, pl.ds(NS * CH0 + sid * CH1, CH1)],
                        idx_v.at[pl.ds(0, CH1)])

    zv = jnp.full((16,), 0.0, jnp.float32)

    def zrow(r, _):
        hist_v[pl.ds(r * 16, 16)] = zv
        return 0

    lax.fori_loop(0, NPAD // 16, zrow, 0)

    ones = jnp.full((16,), 1.0, jnp.float32)

    def chunk(j, _):
        for k in range(K // 16):
            idx = idx_v[j, pl.ds(k * 16, 16)]
            plsc.addupdate_scatter(hist_v, [idx], ones)
        return 0

    @pl.when(cid == 0)
    def _():
        lax.fori_loop(0, CH0, chunk, 0)

    @pl.when(cid != 0)
    def _():
        lax.fori_loop(0, CH1, chunk, 0)

    pltpu.sync_copy(hist_v, hsum.at[sid])
    plsc.subcore_barrier()

    base = sid * ROWS_PER_TILE
    pltpu.sync_copy(hsum.at[:, pl.ds(base, ROWS_PER_TILE)], tb)

    def red(c, _):
        acc = tb[0, pl.ds(c * 16, 16)]
        for t in range(1, NS):
            acc = acc + tb[t, pl.ds(c * 16, 16)]
        res_v[pl.ds(c * 16, 16)] = acc
        return 0

    lax.fori_loop(0, ROWS_PER_TILE // 16, red, 0)
    pltpu.sync_copy(res_v, out_hbm.at[cid, pl.ds(base, ROWS_PER_TILE)])


@functools.cache
def _hist_kernel():
    mesh = plsc.VectorSubcoreMesh(core_axis_name="c", subcore_axis_name="s")
    return pl.kernel(
        _hist_body,
        out_type=jax.ShapeDtypeStruct((NC, NPAD), jnp.float32),
        mesh=mesh,
        compiler_params=pltpu.CompilerParams(needs_layout_passes=False, use_tc_tiling_on_sc=False),
        scratch_types=[
            pltpu.VMEM((CHMAX, K), jnp.int32),
            pltpu.VMEM((NPAD,), jnp.float32),
            pltpu.VMEM_SHARED((NS, NPAD), jnp.float32),
            pltpu.VMEM((NS, ROWS_PER_TILE), jnp.float32),
            pltpu.VMEM((ROWS_PER_TILE,), jnp.float32),
        ],
    )


def _edge_body(y_hbm, e_hbm, out_hbm,
               src_v, dst_v, rows, acc, gsems):
    cid = lax.axis_index("c")
    sid = lax.axis_index("s")

    @pl.when(cid == 0)
    def _():
        start = sid * CH0
        pltpu.sync_copy(e_hbm.at[0, pl.ds(start, CH0)],
                        src_v.at[pl.ds(0, CH0)])
        pltpu.sync_copy(e_hbm.at[1, pl.ds(start, CH0)],
                        dst_v.at[pl.ds(0, CH0)])

    @pl.when(cid != 0)
    def _():
        start = NS * CH0 + sid * CH1
        pltpu.sync_copy(e_hbm.at[0, pl.ds(start, CH1)],
                        src_v.at[pl.ds(0, CH1)])
        pltpu.sync_copy(e_hbm.at[1, pl.ds(start, CH1)],
                        dst_v.at[pl.ds(0, CH1)])

    zv = jnp.full((16,), 0.0, jnp.float32)

    def zrow(r, _):
        for c in range(D // 16):
            rows[0, r, pl.ds(c * 16, 16)] = zv
        return 0

    lax.fori_loop(0, K, zrow, 0)
    base = sid * ROWS_PER_TILE
    for s in range(ROWS_PER_TILE // K):
        pltpu.sync_copy(rows.at[0], acc.at[pl.ds(base + s * K, K)])
    plsc.subcore_barrier()

    pltpu.async_copy(y_hbm.at[src_v.at[0]], rows.at[0], gsems.at[0])

    def run(nch):
        def step(i, _):
            j = 2 * i
            pltpu.make_async_copy(y_hbm.at[src_v.at[j]], rows.at[0],
                                  gsems.at[0]).wait()
            pltpu.async_copy(y_hbm.at[src_v.at[j + 1]], rows.at[1],
                             gsems.at[1])
            pltpu.sync_copy(rows.at[0], acc.at[dst_v.at[j]], add=True)
            pltpu.make_async_copy(y_hbm.at[src_v.at[j + 1]], rows.at[1],
                                  gsems.at[1]).wait()

            @pl.when(i < nch // 2 - 1)
            def _():
                pltpu.async_copy(y_hbm.at[src_v.at[j + 2]], rows.at[0],
                                 gsems.at[0])

            pltpu.sync_copy(rows.at[1], acc.at[dst_v.at[j + 1]], add=True)
            return 0

        lax.fori_loop(0, nch // 2, step, 0)

    @pl.when(cid == 0)
    def _():
        run(CH0)

    @pl.when(cid != 0)
    def _():
        run(CH1)

    plsc.subcore_barrier()
    pltpu.sync_copy(acc.at[pl.ds(base, ROWS_PER_TILE)],
                    out_hbm.at[cid, pl.ds(base, ROWS_PER_TILE)])


@functools.cache
def _edge_kernel():
    mesh = plsc.VectorSubcoreMesh(core_axis_name="c", subcore_axis_name="s")
    return pl.kernel(
        _edge_body,
        out_type=jax.ShapeDtypeStruct((NC, NPAD, D), jnp.float32),
        mesh=mesh,
        compiler_params=pltpu.CompilerParams(needs_layout_passes=False, use_tc_tiling_on_sc=False),
        scratch_types=[
            pltpu.VMEM((CHMAX, K), jnp.int32),
            pltpu.VMEM((CHMAX, K), jnp.int32),
            pltpu.VMEM((2, K, D), jnp.float32),
            pltpu.VMEM_SHARED((NPAD, D), jnp.float32),
            pltpu.SemaphoreType.DMA((2,)),
        ],
    )


def _prep_body(state_ref, w_ref, hist_ref, y_ref, dinv_ref):
    deg = hist_ref[0, :] + hist_ref[1, :] + 1.0
    dinv = lax.rsqrt(deg)[:N].reshape(N, 1)
    dinv_ref[...] = dinv
    xw = jnp.dot(state_ref[...], w_ref[...], preferred_element_type=jnp.float32)
    y_ref[...] = xw * dinv


def _tc_prep(state, w_conv, hist):
    return pl.pallas_call(
        _prep_body,
        out_shape=[
            jax.ShapeDtypeStruct((N, D), jnp.float32),
            jax.ShapeDtypeStruct((N, 1), jnp.float32),
        ],
    )(state, w_conv, hist)


def _final_body(agg_ref, y_ref, dinv_ref, state_ref, bc_ref,
                w1_ref, b1_ref, w2_ref, b2_ref, w3_ref, b3_ref, out_ref):
    agg = agg_ref[0, :N, :] + agg_ref[1, :N, :]
    conv = dinv_ref[...] * (agg + y_ref[...]) + bc_ref[...]
    h = jnp.maximum(conv, 0.0) + state_ref[...]
    z = jnp.dot(h, w1_ref[...], preferred_element_type=jnp.float32) + b1_ref[...]
    z = jnp.where(z >= 0.0, z, 0.01 * z)
    z = jnp.dot(z, w2_ref[...], preferred_element_type=jnp.float32) + b2_ref[...]
    z = jnp.where(z >= 0.0, z, 0.01 * z)
    t = jnp.sum(z * w3_ref[...], axis=1, keepdims=True) + b3_ref[...]
    c = jnp.maximum(t, 0.0) + jnp.log1p(jnp.exp(-jnp.abs(t)))
    out_ref[...] = c / (jnp.sum(c) + 1e-20)


def _tc_final(agg2, y, dinv, state, b_conv, W1, b1, W2, b2, w3r, b3):
    return pl.pallas_call(
        _final_body,
        out_shape=jax.ShapeDtypeStruct((N, 1), jnp.float32),
    )(agg2, y, dinv, state, b_conv, W1, b1, W2, b2, w3r, b3)


def kernel(state, edge_index, W_conv, b_conv, W1, b1, W2, b2, W3, b3,
           deterministic=True):
    e3 = edge_index.reshape(2, NCH_TOTAL, K)

    hist = _hist_kernel()(e3)
    y, dinv = _tc_prep(state, W_conv, hist)
    agg2 = _edge_kernel()(y, e3)
    action = _tc_final(agg2, y, dinv, state,
                       b_conv.reshape(1, D),
                       W1, b1.reshape(1, -1), W2, b2.reshape(1, -1),
                       W3.reshape(1, -1), b3.reshape(1, 1))
    return action.reshape(N // 10, 10)
